# baseline (speedup 1.0000x reference)
import functools

import jax
import jax.numpy as jnp
import numpy as np
from jax.experimental import pallas as pl
from jax.experimental.pallas import tpu as pltpu

_CHUNK = 128
_NH = 4


def _cumsum_kernel(block_w, x_hbm, t_ref, m_ref, o_hbm, carry_ref, g_ref,
                   s_ref, xbuf, obuf, insem, outsem):
    k = pl.program_id(0)
    nsteps = pl.num_programs(0)
    nch = block_w // _CHUNK
    c_row = nch
    sw = block_w // _NH
    sch = nch // _NH
    par = jax.lax.rem(k, 2)
    nxt = jax.lax.rem(k + 1, 2)

    def _in_copy(step, buf, h):
        return pltpu.make_async_copy(
            x_hbm.at[:, pl.ds(step * block_w + h * sw, sw)],
            xbuf.at[buf, h],
            insem.at[buf, h])

    def _out_copy(step, h):
        return pltpu.make_async_copy(
            obuf.at[:, pl.ds(h * sw, sw)],
            o_hbm.at[:, pl.ds(step * block_w + h * sw, sw)],
            outsem.at[h])

    @pl.when(k == 0)
    def _():
        for h in range(_NH):
            _in_copy(0, 0, h).start()
        carry_ref[...] = jnp.zeros_like(carry_ref)
        gi = jax.lax.broadcasted_iota(jnp.int32, (block_w, _CHUNK), 0)
        gc = jax.lax.broadcasted_iota(jnp.int32, (block_w, _CHUNK), 1)
        g_ref[...] = ((gi // _CHUNK) == gc).astype(jnp.bfloat16)
        sd = jax.lax.broadcasted_iota(jnp.int32, (_CHUNK, block_w + _CHUNK), 0)
        sj = jax.lax.broadcasted_iota(jnp.int32, (_CHUNK, block_w + _CHUNK), 1)
        s_ref[...] = ((sd < jnp.minimum(sj // _CHUNK, nch))
                      | (sd == c_row)).astype(jnp.bfloat16)

    @pl.when(k + 1 < nsteps)
    def _():
        for h in range(_NH):
            _in_copy(k + 1, nxt, h).start()

    t = t_ref[...]
    ctf_f32 = carry_ref[...] * m_ref[...]
    for h in range(_NH):
        _in_copy(k, par, h).wait()
        xh = xbuf.at[par, h]
        ct_h = jax.lax.dot(xh[...].astype(jnp.bfloat16),
                           g_ref[h * sw:(h + 1) * sw],
                           preferred_element_type=jnp.float32)
        ctf_f32 = ctf_f32 + ct_h
        ctf = ctf_f32.astype(jnp.bfloat16)
        if h == _NH - 1:
            carry_ref[...] = jax.lax.dot(
                ctf, s_ref[:, block_w:block_w + _CHUNK],
                preferred_element_type=jnp.float32)

        @pl.when(k > 0)
        def _():
            _out_copy(k - 1, h).wait()

        for c in range(sch):
            gc_ = h * sch + c
            sl = slice(gc_ * _CHUNK, (gc_ + 1) * _CHUNK)
            lsl = slice(c * _CHUNK, (c + 1) * _CHUNK)
            local = jax.lax.dot(xh[:, lsl].astype(jnp.bfloat16), t,
                                preferred_element_type=jnp.float32)
            carr = jax.lax.dot(ctf, s_ref[:, sl],
                               preferred_element_type=jnp.float32)
            obuf[:, sl] = local + carr
        _out_copy(k, h).start()

    @pl.when(k == nsteps - 1)
    def _():
        for h in range(_NH):
            _out_copy(k, h).wait()


@jax.jit
def kernel(x):
    rows, n = x.shape
    block_w = 8192
    nch = block_w // _CHUNK
    tri = jnp.asarray(np.triu(np.ones((_CHUNK, _CHUNK), np.float32)),
                      dtype=jnp.bfloat16)
    m = np.zeros((_CHUNK, _CHUNK), np.float32)
    m[:, nch] = 1.0
    m = jnp.asarray(m)
    return pl.pallas_call(
        functools.partial(_cumsum_kernel, block_w),
        grid=(n // block_w,),
        in_specs=[
            pl.BlockSpec(memory_space=pltpu.MemorySpace.HBM),
            pl.BlockSpec((_CHUNK, _CHUNK), lambda k: (0, 0)),
            pl.BlockSpec((_CHUNK, _CHUNK), lambda k: (0, 0)),
        ],
        out_specs=pl.BlockSpec(memory_space=pltpu.MemorySpace.HBM),
        out_shape=jax.ShapeDtypeStruct((rows, n), jnp.float32),
        scratch_shapes=[
            pltpu.VMEM((rows, _CHUNK), jnp.float32),
            pltpu.VMEM((block_w, _CHUNK), jnp.bfloat16),
            pltpu.VMEM((_CHUNK, block_w + _CHUNK), jnp.bfloat16),
            pltpu.VMEM((2, _NH, rows, block_w // _NH), jnp.float32),
            pltpu.VMEM((rows, block_w), jnp.float32),
            pltpu.SemaphoreType.DMA((2, _NH)),
            pltpu.SemaphoreType.DMA((_NH,)),
        ],
    )(x, tri, m)


# speedup vs baseline: 1.1231x; 1.1231x over previous
import functools

import jax
import jax.numpy as jnp
import numpy as np
from jax.experimental import pallas as pl
from jax.experimental.pallas import tpu as pltpu

_CHUNK = 128


def _cumsum_kernel(block_w, x_hbm, t_ref, m_ref, o_hbm, carry_ref, g_ref,
                   s_ref, xbuf, obuf, insem, outsem):
    k = pl.program_id(0)
    nsteps = pl.num_programs(0)
    nch = block_w // _CHUNK
    c_row = nch
    hw = block_w // 2
    hch = nch // 2
    par = jax.lax.rem(k, 2)
    nxt = jax.lax.rem(k + 1, 2)

    def _in_copy(step, buf, h):
        return pltpu.make_async_copy(
            x_hbm.at[:, pl.ds(step * block_w + h * hw, hw)],
            xbuf.at[buf, h],
            insem.at[buf, h])

    def _out_copy(step, h):
        return pltpu.make_async_copy(
            obuf.at[:, pl.ds(h * hw, hw)],
            o_hbm.at[:, pl.ds(step * block_w + h * hw, hw)],
            outsem.at[h])

    @pl.when(k == 0)
    def _():
        _in_copy(0, 0, 0).start()
        _in_copy(0, 0, 1).start()
        carry_ref[...] = jnp.zeros_like(carry_ref)
        gi = jax.lax.broadcasted_iota(jnp.int32, (block_w, _CHUNK), 0)
        gc = jax.lax.broadcasted_iota(jnp.int32, (block_w, _CHUNK), 1)
        g_ref[...] = ((gi // _CHUNK) == gc).astype(jnp.bfloat16)
        sd = jax.lax.broadcasted_iota(jnp.int32, (_CHUNK, block_w + _CHUNK), 0)
        sj = jax.lax.broadcasted_iota(jnp.int32, (_CHUNK, block_w + _CHUNK), 1)
        s_ref[...] = ((sd < jnp.minimum(sj // _CHUNK, nch))
                      | (sd == c_row)).astype(jnp.bfloat16)

    @pl.when(k + 1 < nsteps)
    def _():
        _in_copy(k + 1, nxt, 0).start()
        _in_copy(k + 1, nxt, 1).start()

    t = t_ref[...]

    _in_copy(k, par, 0).wait()
    xa = xbuf.at[par, 0]
    ct_a = jax.lax.dot(xa[...].astype(jnp.bfloat16), g_ref[:hw],
                       preferred_element_type=jnp.float32)
    ctf_a = ct_a + carry_ref[...] * m_ref[...]
    ctf_ab = ctf_a.astype(jnp.bfloat16)

    @pl.when(k > 0)
    def _():
        _out_copy(k - 1, 0).wait()

    for c in range(hch):
        sl = slice(c * _CHUNK, (c + 1) * _CHUNK)
        local = jax.lax.dot(xa[:, sl].astype(jnp.bfloat16), t,
                            preferred_element_type=jnp.float32)
        carr = jax.lax.dot(ctf_ab, s_ref[:, sl],
                           preferred_element_type=jnp.float32)
        obuf[:, sl] = local + carr
    _out_copy(k, 0).start()

    _in_copy(k, par, 1).wait()
    xb = xbuf.at[par, 1]
    ct_b = jax.lax.dot(xb[...].astype(jnp.bfloat16), g_ref[hw:],
                       preferred_element_type=jnp.float32)
    ctf = (ctf_a + ct_b).astype(jnp.bfloat16)
    carry_ref[...] = jax.lax.dot(ctf, s_ref[:, block_w:block_w + _CHUNK],
                                 preferred_element_type=jnp.float32)

    @pl.when(k > 0)
    def _():
        _out_copy(k - 1, 1).wait()

    for c in range(hch):
        sl = slice(c * _CHUNK, (c + 1) * _CHUNK)
        local = jax.lax.dot(xb[:, sl].astype(jnp.bfloat16), t,
                            preferred_element_type=jnp.float32)
        carr = jax.lax.dot(ctf, s_ref[:, (hch + c) * _CHUNK:
                                      (hch + c + 1) * _CHUNK],
                           preferred_element_type=jnp.float32)
        obuf[:, hw + c * _CHUNK:hw + (c + 1) * _CHUNK] = local + carr
    _out_copy(k, 1).start()

    @pl.when(k == nsteps - 1)
    def _():
        _out_copy(k, 0).wait()
        _out_copy(k, 1).wait()


@jax.jit
def kernel(x):
    rows, n = x.shape
    block_w = 8192
    nch = block_w // _CHUNK
    tri = jnp.asarray(np.triu(np.ones((_CHUNK, _CHUNK), np.float32)),
                      dtype=jnp.bfloat16)
    m = np.zeros((_CHUNK, _CHUNK), np.float32)
    m[:, nch] = 1.0
    m = jnp.asarray(m)
    return pl.pallas_call(
        functools.partial(_cumsum_kernel, block_w),
        grid=(n // block_w,),
        in_specs=[
            pl.BlockSpec(memory_space=pltpu.MemorySpace.HBM),
            pl.BlockSpec((_CHUNK, _CHUNK), lambda k: (0, 0)),
            pl.BlockSpec((_CHUNK, _CHUNK), lambda k: (0, 0)),
        ],
        out_specs=pl.BlockSpec(memory_space=pltpu.MemorySpace.HBM),
        out_shape=jax.ShapeDtypeStruct((rows, n), jnp.float32),
        scratch_shapes=[
            pltpu.VMEM((rows, _CHUNK), jnp.float32),
            pltpu.VMEM((block_w, _CHUNK), jnp.bfloat16),
            pltpu.VMEM((_CHUNK, block_w + _CHUNK), jnp.bfloat16),
            pltpu.VMEM((2, 2, rows, block_w // 2), jnp.float32),
            pltpu.VMEM((rows, block_w), jnp.float32),
            pltpu.SemaphoreType.DMA((2, 2)),
            pltpu.SemaphoreType.DMA((2,)),
        ],
    )(x, tri, m)
